# Initial kernel scaffold; baseline (speedup 1.0000x reference)
import jax
import jax.numpy as jnp
from jax.experimental import pallas as pl


def kernel(x, edge_index, W_in, W1, b1, W2, b2):
    raise NotImplementedError("write your pallas kernel here")


# speedup vs baseline: 6.3111x; 6.3111x over previous
import functools

import jax
import jax.numpy as jnp
from jax import lax
from jax.experimental import pallas as pl
from jax.experimental.pallas import tpu as pltpu
from jax.experimental.pallas import tpu_sc as plsc

_N = 10000
_E = 160000
_A = 64
_D = 256
_ITERS = 4

_NS = 16
_N2 = 10240
_ROWS = _N2 // _NS
_EPT = _E // _NS
_K = 80
_NCH = _EPT // _K


def _mesh():
    return plsc.VectorSubcoreMesh(core_axis_name="c", subcore_axis_name="s")


_KD = 40
_EPT_D = _E // 32
_NCH_D = _EPT_D // _KD


@functools.partial(
    pl.kernel,
    out_type=(jax.ShapeDtypeStruct((_N2, 128), jnp.float32),
              jax.ShapeDtypeStruct((_N2, 128), jnp.float32)),
    mesh=_mesh(),
    scratch_types=[
        pltpu.VMEM_SHARED((_N2, 128), jnp.float32),
        pltpu.VMEM((_KD,), jnp.int32),
        pltpu.VMEM((_KD, 128), jnp.float32),
    ],
)
def _sc_deg(dst_hbm, zeros_hbm, ones_hbm, outa_hbm, outb_hbm,
            sp_deg, dst_v, ones_v):
    c = lax.axis_index("c")
    s = lax.axis_index("s")
    pltpu.sync_copy(zeros_hbm.at[pl.ds(s * _ROWS, _ROWS)],
                    sp_deg.at[pl.ds(s * _ROWS, _ROWS)])
    pltpu.sync_copy(ones_hbm, ones_v)
    plsc.subcore_barrier()

    def body(i, carry):
        base = (c * _NS + s) * _EPT_D + i * _KD
        pltpu.sync_copy(dst_hbm.at[pl.ds(base, _KD)], dst_v)
        pltpu.sync_copy(ones_v, sp_deg.at[dst_v], add=True)
        return carry

    lax.fori_loop(0, _NCH_D, body, 0)
    plsc.subcore_barrier()

    @pl.when(c == 0)
    def _():
        pltpu.sync_copy(sp_deg.at[pl.ds(s * _ROWS, _ROWS)],
                        outa_hbm.at[pl.ds(s * _ROWS, _ROWS)])

    @pl.when(c == 1)
    def _():
        pltpu.sync_copy(sp_deg.at[pl.ds(s * _ROWS, _ROWS)],
                        outb_hbm.at[pl.ds(s * _ROWS, _ROWS)])


def _make_sc_agg(dc):
    @functools.partial(
        pl.kernel,
        out_type=(jax.ShapeDtypeStruct((_N2, dc), jnp.float32),
                  jax.ShapeDtypeStruct((_N2, dc), jnp.float32)),
        mesh=_mesh(),
        scratch_types=[
            pltpu.VMEM_SHARED((_N2, dc), jnp.float32),
            pltpu.VMEM((_K,), jnp.int32),
            pltpu.VMEM((_K,), jnp.int32),
            pltpu.VMEM((_K, dc), jnp.float32),
            pltpu.SemaphoreType.DMA,
        ],
    )
    def agg(zpa_hbm, zpb_hbm, src_hbm, dst_hbm, outa_hbm, outb_hbm,
            sp_agg, src_v, dst_v, rows_v, sem):
        c = lax.axis_index("c")
        s = lax.axis_index("s")

        @pl.when(c == 0)
        def _():
            pltpu.sync_copy(zpa_hbm.at[pl.ds(s * _ROWS, _ROWS)],
                            sp_agg.at[pl.ds(s * _ROWS, _ROWS)])

        @pl.when(c == 1)
        def _():
            pltpu.sync_copy(zpb_hbm.at[pl.ds(s * _ROWS, _ROWS)],
                            sp_agg.at[pl.ds(s * _ROWS, _ROWS)])

        plsc.subcore_barrier()

        def body(i, carry):
            base = s * _EPT + i * _K
            pltpu.sync_copy(src_hbm.at[pl.ds(base, _K)], src_v)
            pltpu.sync_copy(dst_hbm.at[pl.ds(base, _K)], dst_v)

            @pl.when(c == 0)
            def _():
                pltpu.async_copy(zpa_hbm.at[src_v], rows_v, sem).wait()

            @pl.when(c == 1)
            def _():
                pltpu.async_copy(zpb_hbm.at[src_v], rows_v, sem).wait()

            pltpu.sync_copy(rows_v, sp_agg.at[dst_v], add=True)
            return carry

        lax.fori_loop(0, _NCH, body, 0)
        plsc.subcore_barrier()

        @pl.when(c == 0)
        def _():
            pltpu.sync_copy(sp_agg.at[pl.ds(s * _ROWS, _ROWS)],
                            outa_hbm.at[pl.ds(s * _ROWS, _ROWS)])

        @pl.when(c == 1)
        def _():
            pltpu.sync_copy(sp_agg.at[pl.ds(s * _ROWS, _ROWS)],
                            outb_hbm.at[pl.ds(s * _ROWS, _ROWS)])

    return agg


_sc_agg128 = _make_sc_agg(_D // 2)

_KF = 40
_EPT_F = _E // 32
_NCH_F = _EPT_F // _KF


@functools.partial(
    pl.kernel,
    out_type=(jax.ShapeDtypeStruct((_N2, 128), jnp.float32),
              jax.ShapeDtypeStruct((_N2, 128), jnp.float32)),
    mesh=_mesh(),
    scratch_types=[
        pltpu.VMEM_SHARED((_N2, 128), jnp.float32),
        pltpu.VMEM((_KF,), jnp.int32),
        pltpu.VMEM((_KF,), jnp.int32),
        pltpu.VMEM((_KF, 128), jnp.float32),
        pltpu.SemaphoreType.DMA,
    ],
)
def _sc_aggf(zpf_hbm, zeros_hbm, src_hbm, dst_hbm, outa_hbm, outb_hbm,
             sp_agg, src_v, dst_v, rows_v, sem):
    c = lax.axis_index("c")
    s = lax.axis_index("s")

    @pl.when(c == 0)
    def _():
        pltpu.sync_copy(zpf_hbm.at[pl.ds(s * _ROWS, _ROWS)],
                        sp_agg.at[pl.ds(s * _ROWS, _ROWS)])

    @pl.when(c == 1)
    def _():
        pltpu.sync_copy(zeros_hbm.at[pl.ds(s * _ROWS, _ROWS)],
                        sp_agg.at[pl.ds(s * _ROWS, _ROWS)])

    plsc.subcore_barrier()

    def body(i, carry):
        base = (c * _NS + s) * _EPT_F + i * _KF
        pltpu.sync_copy(src_hbm.at[pl.ds(base, _KF)], src_v)
        pltpu.sync_copy(dst_hbm.at[pl.ds(base, _KF)], dst_v)
        pltpu.async_copy(zpf_hbm.at[src_v], rows_v, sem).wait()
        pltpu.sync_copy(rows_v, sp_agg.at[dst_v], add=True)
        return carry

    lax.fori_loop(0, _NCH_F, body, 0)
    plsc.subcore_barrier()

    @pl.when(c == 0)
    def _():
        pltpu.sync_copy(sp_agg.at[pl.ds(s * _ROWS, _ROWS)],
                        outa_hbm.at[pl.ds(s * _ROWS, _ROWS)])

    @pl.when(c == 1)
    def _():
        pltpu.sync_copy(sp_agg.at[pl.ds(s * _ROWS, _ROWS)],
                        outb_hbm.at[pl.ds(s * _ROWS, _ROWS)])


def _tc_first_body(x_ref, dega_ref, degb_ref, win_ref, w1_ref,
                   zpa_ref, zpb_ref, dinv_ref):
    deg = dega_ref[:, :1] + degb_ref[:, :1] + 1.0
    dinv = lax.rsqrt(deg)
    onehot = (x_ref[...] ==
              lax.broadcasted_iota(jnp.int32, (1, _A), 1)).astype(jnp.float32)
    h0 = jnp.dot(onehot, win_ref[...], preferred_element_type=jnp.float32)
    z = jnp.dot(h0, w1_ref[...], preferred_element_type=jnp.float32)
    zp = z * dinv
    zpa_ref[...] = zp[:, : _D // 2]
    zpb_ref[...] = zp[:, _D // 2:]
    dinv_ref[...] = dinv


_tc_first = pl.pallas_call(
    _tc_first_body,
    out_shape=(jax.ShapeDtypeStruct((_N2, _D // 2), jnp.float32),
               jax.ShapeDtypeStruct((_N2, _D // 2), jnp.float32),
               jax.ShapeDtypeStruct((_N2, 1), jnp.float32)),
)


def _tc_epi_body(agga_ref, aggb_ref, dinv_ref, b1_ref, w1_ref,
                 zpa_ref, zpb_ref):
    dinv = dinv_ref[...]
    agg = jnp.concatenate([agga_ref[...], aggb_ref[...]], axis=1)
    h = jnp.maximum(agg * dinv + b1_ref[...][None, :], 0.0)
    z = jnp.dot(h, w1_ref[...], preferred_element_type=jnp.float32)
    zp = z * dinv
    zpa_ref[...] = zp[:, : _D // 2]
    zpb_ref[...] = zp[:, _D // 2:]


_tc_epi = pl.pallas_call(
    _tc_epi_body,
    out_shape=(jax.ShapeDtypeStruct((_N2, _D // 2), jnp.float32),
               jax.ShapeDtypeStruct((_N2, _D // 2), jnp.float32)),
)


def _tc_epi_final_body(agga_ref, aggb_ref, dinv_ref, b1_ref, w2_ref,
                       zpf_ref):
    dinv = dinv_ref[...]
    agg = jnp.concatenate([agga_ref[...], aggb_ref[...]], axis=1)
    h = jnp.maximum(agg * dinv + b1_ref[...][None, :], 0.0)
    z = jnp.dot(h, w2_ref[...], preferred_element_type=jnp.float32)
    zp = z * dinv
    zpf_ref[...] = jnp.pad(zp, ((0, 0), (0, 128 - _A)))


_tc_epi_final = pl.pallas_call(
    _tc_epi_final_body,
    out_shape=jax.ShapeDtypeStruct((_N2, 128), jnp.float32),
)


def _tc_final_body(agga_ref, aggb_ref, dinv_ref, b2_ref, out_ref):
    agg = agga_ref[...] + aggb_ref[...]
    out_ref[...] = agg[:, :_A] * dinv_ref[...] + b2_ref[...][None, :]


_tc_final = pl.pallas_call(
    _tc_final_body,
    out_shape=jax.ShapeDtypeStruct((_N2, _A), jnp.float32),
)


def kernel(x, edge_index, W_in, W1, b1, W2, b2):
    src = edge_index[0]
    dst = edge_index[1]
    zeros_n128 = jnp.zeros((_N2, 128), jnp.float32)
    x_p = jnp.pad(x, ((0, _N2 - _N), (0, 0)))

    ones_kd = jnp.ones((_KD, 128), jnp.float32)
    dega, degb = _sc_deg(dst, zeros_n128, ones_kd)
    zp_a, zp_b, dinv = _tc_first(x_p, dega, degb, W_in, W1)
    agg_a, agg_b = _sc_agg128(zp_a, zp_b, src, dst)
    for _ in range(_ITERS - 1):
        zp_a, zp_b = _tc_epi(agg_a, agg_b, dinv, b1, W1)
        agg_a, agg_b = _sc_agg128(zp_a, zp_b, src, dst)
    zpf = _tc_epi_final(agg_a, agg_b, dinv, b1, W2)
    aggf_a, aggf_b = _sc_aggf(zpf, zeros_n128, src, dst)
    return _tc_final(aggf_a, aggf_b, dinv, b2)[:_N]


# speedup vs baseline: 7.9185x; 1.2547x over previous
import functools

import jax
import jax.numpy as jnp
from jax import lax
from jax.experimental import pallas as pl
from jax.experimental.pallas import tpu as pltpu
from jax.experimental.pallas import tpu_sc as plsc

_N = 10000
_E = 160000
_A = 64
_D = 256
_ITERS = 4

_NS = 16
_N2 = 10240
_ROWS = _N2 // _NS
_K = 128
_NCH = 80
_NCHD = 40
_PAD_NODE = _N


def _mesh():
    return plsc.VectorSubcoreMesh(core_axis_name="c", subcore_axis_name="s")


@functools.partial(
    pl.kernel,
    out_type=jax.ShapeDtypeStruct((2 * _N2, 128), jnp.float32),
    mesh=_mesh(),
    scratch_types=[
        pltpu.VMEM_SHARED((_N2, 128), jnp.float32),
        pltpu.VMEM((_K,), jnp.int32),
        pltpu.VMEM((_K,), jnp.int32),
        pltpu.VMEM((_K, 128), jnp.float32),
        pltpu.SemaphoreType.DMA,
        pltpu.SemaphoreType.DMA,
        pltpu.SemaphoreType.DMA,
        pltpu.SemaphoreType.DMA,
    ],
)
def _sc_deg(dst2_hbm, zeros_hbm, ones_hbm, deg_hbm,
            sp_deg, d0, d1, ones_v, dm0, dm1, sm0, sm1):
    c = lax.axis_index("c")
    s = lax.axis_index("s")
    pltpu.sync_copy(zeros_hbm.at[pl.ds(s * _ROWS, _ROWS)],
                    sp_deg.at[pl.ds(s * _ROWS, _ROWS)])
    pltpu.sync_copy(ones_hbm, ones_v)
    plsc.subcore_barrier()

    row0 = (c * _NS + s) * _NCHD
    dbufs = (d0, d1)
    dsems = (dm0, dm1)
    ssems = (sm0, sm1)

    def load_dst(j, db, sem):
        pltpu.async_copy(dst2_hbm.at[row0 + j], db, sem)

    def wait_dst(j, db, sem):
        pltpu.make_async_copy(dst2_hbm.at[row0 + j], db, sem).wait()

    def issue_s(db, sem):
        pltpu.async_copy(ones_v, sp_deg.at[db], sem, add=True)

    def wait_s(db, sem):
        pltpu.make_async_copy(ones_v, sp_deg.at[db], sem).wait()

    load_dst(0, d0, dm0)
    load_dst(1, d1, dm1)

    def body(jj, carry):
        for b in range(2):
            j = 2 * jj + b
            wait_dst(j, dbufs[b], dsems[b])
            issue_s(dbufs[b], ssems[b])
            wait_s(dbufs[b], ssems[b])

            @pl.when(j + 2 < _NCHD)
            def _():
                load_dst(j + 2, dbufs[b], dsems[b])
        return carry

    lax.fori_loop(0, _NCHD // 2, body, 0)
    plsc.subcore_barrier()
    pltpu.sync_copy(sp_deg.at[pl.ds(s * _ROWS, _ROWS)],
                    deg_hbm.at[pl.ds(c * _N2 + s * _ROWS, _ROWS)])


def _make_sc_agg(ncht, core_split):
    nsl = ncht * _K

    @functools.partial(
        pl.kernel,
        out_type=jax.ShapeDtypeStruct((2 * _N2, 128), jnp.float32),
        mesh=_mesh(),
        scratch_types=[
            pltpu.VMEM_SHARED((_N2, 128), jnp.float32),
            pltpu.VMEM((nsl,), jnp.int32),
            pltpu.VMEM((_K,), jnp.int32),
            pltpu.VMEM((_K,), jnp.int32),
            pltpu.VMEM((_K, 128), jnp.float32),
            pltpu.VMEM((_K, 128), jnp.float32),
            pltpu.SemaphoreType.DMA,
            pltpu.SemaphoreType.DMA,
            pltpu.SemaphoreType.DMA,
            pltpu.SemaphoreType.DMA,
            pltpu.SemaphoreType.DMA,
            pltpu.SemaphoreType.DMA,
        ],
    )
    def agg(zp_hbm, init_hbm, srca_hbm, srcb_hbm, dst2_hbm, out_hbm,
            sp_agg, srcbuf, d0, d1, r0, r1, g0, g1, s0, s1, dm0, dm1):
        c = lax.axis_index("c")
        s = lax.axis_index("s")

        pltpu.sync_copy(init_hbm.at[pl.ds(c * _N2 + s * _ROWS, _ROWS)],
                        sp_agg.at[pl.ds(s * _ROWS, _ROWS)])
        tile = c * _NS + s if core_split else s

        @pl.when(c == 0)
        def _():
            pltpu.sync_copy(srca_hbm.at[pl.ds(tile * nsl, nsl)], srcbuf)

        @pl.when(c == 1)
        def _():
            pltpu.sync_copy(srcb_hbm.at[pl.ds(tile * nsl, nsl)], srcbuf)

        plsc.subcore_barrier()

        row0 = tile * ncht
        dbufs = (d0, d1)
        rbufs = (r0, r1)
        gsems = (g0, g1)
        ssems = (s0, s1)
        dsems = (dm0, dm1)

        def load_dst(j, db, sem):
            pltpu.async_copy(dst2_hbm.at[row0 + j], db, sem)

        def wait_dst(j, db, sem):
            pltpu.make_async_copy(dst2_hbm.at[row0 + j], db, sem).wait()

        def issue_g(j, buf, sem):
            pltpu.async_copy(zp_hbm.at[srcbuf.at[pl.ds(j * _K, _K)]], buf, sem)

        def wait_g(j, buf, sem):
            pltpu.make_async_copy(
                zp_hbm.at[srcbuf.at[pl.ds(j * _K, _K)]], buf, sem).wait()

        def issue_s(buf, db, sem):
            pltpu.async_copy(buf, sp_agg.at[db], sem, add=True)

        def wait_s(buf, db, sem):
            pltpu.make_async_copy(buf, sp_agg.at[db], sem).wait()

        load_dst(0, d0, dm0)
        load_dst(1, d1, dm1)
        issue_g(0, r0, g0)
        issue_g(1, r1, g1)

        def body(jj, carry):
            for b in range(2):
                j = 2 * jj + b
                wait_g(j, rbufs[b], gsems[b])
                wait_dst(j, dbufs[b], dsems[b])
                issue_s(rbufs[b], dbufs[b], ssems[b])
                wait_s(rbufs[b], dbufs[b], ssems[b])

                @pl.when(j + 2 < ncht)
                def _():
                    load_dst(j + 2, dbufs[b], dsems[b])
                    issue_g(j + 2, rbufs[b], gsems[b])
            return carry

        lax.fori_loop(0, ncht // 2, body, 0)
        plsc.subcore_barrier()
        pltpu.sync_copy(sp_agg.at[pl.ds(s * _ROWS, _ROWS)],
                        out_hbm.at[pl.ds(c * _N2 + s * _ROWS, _ROWS)])

    return agg


_sc_agg128 = _make_sc_agg(_NCH, False)
_sc_aggf = _make_sc_agg(_NCHD, True)


def _cat_halves(zcat):
    return jnp.concatenate([zcat[: _N2], zcat[_N2:]], axis=1)


def _tc_first_body(x_ref, deg_ref, win_ref, w1_ref, zp_ref, dinv_ref):
    deg = deg_ref[: _N2, :1] + deg_ref[_N2:, :1] + 1.0
    dinv = lax.rsqrt(deg)
    onehot = (x_ref[...] ==
              lax.broadcasted_iota(jnp.int32, (1, _A), 1)).astype(jnp.float32)
    h0 = jnp.dot(onehot, win_ref[...], preferred_element_type=jnp.float32)
    z = jnp.dot(h0, w1_ref[...], preferred_element_type=jnp.float32)
    zp = z * dinv
    zp_ref[: _N2] = zp[:, : _D // 2]
    zp_ref[_N2:] = zp[:, _D // 2:]
    dinv_ref[...] = dinv


_tc_first = pl.pallas_call(
    _tc_first_body,
    out_shape=(jax.ShapeDtypeStruct((2 * _N2, _D // 2), jnp.float32),
               jax.ShapeDtypeStruct((_N2, 1), jnp.float32)),
)


def _tc_epi_body(aggcat_ref, dinv_ref, b1_ref, w1_ref, zp_ref):
    dinv = dinv_ref[...]
    agg = _cat_halves(aggcat_ref[...])
    h = jnp.maximum(agg * dinv + b1_ref[...][None, :], 0.0)
    z = jnp.dot(h, w1_ref[...], preferred_element_type=jnp.float32)
    zp = z * dinv
    zp_ref[: _N2] = zp[:, : _D // 2]
    zp_ref[_N2:] = zp[:, _D // 2:]


_tc_epi = pl.pallas_call(
    _tc_epi_body,
    out_shape=jax.ShapeDtypeStruct((2 * _N2, _D // 2), jnp.float32),
)


def _tc_epi_final_body(aggcat_ref, dinv_ref, b1_ref, w2_ref, zpf_ref):
    dinv = dinv_ref[...]
    agg = _cat_halves(aggcat_ref[...])
    h = jnp.maximum(agg * dinv + b1_ref[...][None, :], 0.0)
    z = jnp.dot(h, w2_ref[...], preferred_element_type=jnp.float32)
    zp = z * dinv
    zpf_ref[...] = jnp.pad(zp, ((0, 0), (0, 128 - _A)))


_tc_epi_final = pl.pallas_call(
    _tc_epi_final_body,
    out_shape=jax.ShapeDtypeStruct((_N2, 128), jnp.float32),
)


def _tc_final_body(aggf_ref, dinv_ref, b2_ref, out_ref):
    agg = aggf_ref[: _N2] + aggf_ref[_N2:]
    out_ref[...] = agg[:, : _A] * dinv_ref[...] + b2_ref[...][None, :]


_tc_final = pl.pallas_call(
    _tc_final_body,
    out_shape=jax.ShapeDtypeStruct((_N2, _A), jnp.float32),
)


def kernel(x, edge_index, W_in, W1, b1, W2, b2):
    src = edge_index[0]
    dst = edge_index[1]
    x_p = jnp.pad(x, ((0, _N2 - _N), (0, 0)))
    zeros_n128 = jnp.zeros((_N2, 128), jnp.float32)
    ones_kd = jnp.ones((_K, 128), jnp.float32)

    def slots(a, ntiles, ncht):
        a2 = a.reshape(ntiles, _E // ntiles)
        a2 = jnp.pad(a2, ((0, 0), (0, ncht * _K - a2.shape[1])),
                     constant_values=_PAD_NODE)
        return a2.reshape(-1)

    src2a = slots(src, _NS, _NCH)
    src2b = src2a + _N2
    dst2 = slots(dst, _NS, _NCH).reshape(-1, _K)
    src2f = slots(src, 32, _NCHD)
    dst2f = slots(dst, 32, _NCHD).reshape(-1, _K)

    deg = _sc_deg(dst2f, zeros_n128, ones_kd)
    zp, dinv = _tc_first(x_p, deg, W_in, W1)
    agg = _sc_agg128(zp, zp, src2a, src2b, dst2)
    for _ in range(_ITERS - 1):
        zp = _tc_epi(agg, dinv, b1, W1)
        agg = _sc_agg128(zp, zp, src2a, src2b, dst2)
    zpf = _tc_epi_final(agg, dinv, b1, W2)
    zpf2 = jnp.concatenate([zpf, zeros_n128], axis=0)
    aggf = _sc_aggf(zpf, zpf2, src2f, src2f, dst2f)
    return _tc_final(aggf, dinv, b2)[:_N]
